# baseline (speedup 1.0000x reference)
import functools

import jax
import jax.numpy as jnp
from jax import lax
from jax.experimental import pallas as pl
from jax.experimental.pallas import tpu as pltpu
from jax.experimental.pallas import tpu_sc as plsc

_B = 16384
_D = 32
_NC = 2
_NS = 16
_NW = _NC * _NS
_BPW = _B // _NW


def _body(idx_hbm, sp_hbm, su_hbm, out_sp, out_su, idx_v, sem):
    wid = lax.axis_index("s") * _NC + lax.axis_index("c")
    base = wid * _BPW
    pltpu.sync_copy(idx_hbm.at[pl.ds(base, _BPW)], idx_v)

    @plsc.parallel_loop(0, _BPW // 16, 1, unroll=4)
    def issue(g):
        vec = idx_v[pl.ds(g * 16, 16)]
        for l in range(16):
            r = vec[l]
            j = base + g * 16 + l
            pltpu.async_copy(sp_hbm.at[pl.ds(r, 1)], out_sp.at[pl.ds(j, 1)], sem)
            pltpu.async_copy(su_hbm.at[pl.ds(r, 1)], out_su.at[pl.ds(j, 1)], sem)

    def drain(j, carry):
        pltpu.make_async_copy(sp_hbm.at[pl.ds(0, 1)], out_sp.at[pl.ds(base, 1)], sem).wait()
        pltpu.make_async_copy(su_hbm.at[pl.ds(0, 1)], out_su.at[pl.ds(base, 1)], sem).wait()
        return carry

    lax.fori_loop(0, _BPW, drain, 0)


@jax.jit
def kernel(node_indices, B_sp, B_su):
    gather = pl.kernel(
        _body,
        out_type=(
            jax.ShapeDtypeStruct((_B, _D), jnp.float32),
            jax.ShapeDtypeStruct((_B, _D), jnp.float32),
        ),
        mesh=plsc.VectorSubcoreMesh(core_axis_name="c", subcore_axis_name="s"),
        scratch_types=[
            pltpu.VMEM((_BPW,), jnp.int32),
            pltpu.SemaphoreType.DMA,
        ],
        compiler_params=pltpu.CompilerParams(use_tc_tiling_on_sc=True),
    )
    return gather(node_indices.astype(jnp.int32), B_sp, B_su)


# speedup vs baseline: 1.8042x; 1.8042x over previous
import functools

import jax
import jax.numpy as jnp
from jax import lax
from jax.experimental import pallas as pl
from jax.experimental.pallas import tpu as pltpu
from jax.experimental.pallas import tpu_sc as plsc

_B = 16384
_D = 32
_NC = 2
_NS = 16
_NW = _NC * _NS
_BPW = _B // _NW


def _body(idx_hbm, sp_hbm, su_hbm, out_sp, out_su, idx_v, out_buf, sem, osem):
    wid = lax.axis_index("s") * _NC + lax.axis_index("c")
    base = wid * _BPW
    pltpu.sync_copy(idx_hbm.at[pl.ds(base, _BPW)], idx_v)

    for table, out in ((sp_hbm, out_sp), (su_hbm, out_su)):
        def issue(g, carry, table=table):
            vec = idx_v[pl.ds(g * 16, 16)]
            for l in range(16):
                r = vec[l]
                j = g * 16 + l
                pltpu.async_copy(
                    table.at[pl.ds(r, 1)], out_buf.at[pl.ds(j, 1)], sem)
            return carry

        lax.fori_loop(0, _BPW // 16, issue, 0)

        def drain(g, carry, table=table):
            pltpu.make_async_copy(
                table.at[pl.ds(0, 1)], out_buf.at[pl.ds(0, 1)], sem).wait()
            return carry

        lax.fori_loop(0, _BPW, drain, 0)
        pltpu.sync_copy(out_buf, out.at[pl.ds(base, _BPW)])


@jax.jit
def kernel(node_indices, B_sp, B_su):
    gather = pl.kernel(
        _body,
        out_type=(
            jax.ShapeDtypeStruct((_B, _D), jnp.float32),
            jax.ShapeDtypeStruct((_B, _D), jnp.float32),
        ),
        mesh=plsc.VectorSubcoreMesh(core_axis_name="c", subcore_axis_name="s"),
        scratch_types=[
            pltpu.VMEM((_BPW,), jnp.int32),
            pltpu.VMEM((_BPW, _D), jnp.float32),
            pltpu.SemaphoreType.DMA,
            pltpu.SemaphoreType.DMA,
        ],
        compiler_params=pltpu.CompilerParams(use_tc_tiling_on_sc=True),
    )
    return gather(node_indices.astype(jnp.int32), B_sp, B_su)


# speedup vs baseline: 1.8063x; 1.0012x over previous
import functools

import jax
import jax.numpy as jnp
from jax import lax
from jax.experimental import pallas as pl
from jax.experimental.pallas import tpu as pltpu
from jax.experimental.pallas import tpu_sc as plsc

_B = 16384
_D = 32
_NC = 2
_NS = 16
_NW = _NC * _NS
_BPW = _B // _NW


def _body(idx_hbm, sp_hbm, su_hbm, out_sp, out_su, idx_v, out_buf, sem, osem):
    wid = lax.axis_index("s") * _NC + lax.axis_index("c")
    base = wid * _BPW
    pltpu.sync_copy(idx_hbm.at[pl.ds(base, _BPW)], idx_v)

    for table, out in ((sp_hbm, out_sp), (su_hbm, out_su)):
        def issue(g, carry, table=table):
            vec = idx_v[pl.ds(g * 16, 16)]
            for l in range(16):
                r = vec[l]
                j = g * 16 + l
                pltpu.async_copy(
                    table.at[pl.ds(r, 1)], out_buf.at[pl.ds(j, 1)], sem)
            return carry

        lax.fori_loop(0, _BPW // 16, issue, 0)

        def drain(g, carry, table=table):
            pltpu.make_async_copy(
                table.at[pl.ds(0, 1)], out_buf.at[pl.ds(0, 1)], sem).wait()
            return carry

        lax.fori_loop(0, _BPW, drain, 0)
        pltpu.sync_copy(out_buf, out.at[pl.ds(base, _BPW)])


@jax.jit
def kernel(node_indices, B_sp, B_su):
    gather = pl.kernel(
        _body,
        out_type=(
            jax.ShapeDtypeStruct((_B, _D), jnp.float32),
            jax.ShapeDtypeStruct((_B, _D), jnp.float32),
        ),
        mesh=plsc.VectorSubcoreMesh(core_axis_name="c", subcore_axis_name="s"),
        scratch_types=[
            pltpu.VMEM((_BPW,), jnp.int32),
            pltpu.VMEM((_BPW, _D), jnp.float32),
            pltpu.SemaphoreType.DMA,
            pltpu.SemaphoreType.DMA,
        ],
        compiler_params=pltpu.CompilerParams(
            use_tc_tiling_on_sc=True,
            disable_bounds_checks=True,
            disable_semaphore_checks=True,
        ),
    )
    return gather(node_indices.astype(jnp.int32), B_sp, B_su)


# speedup vs baseline: 2.6574x; 1.4712x over previous
import functools

import jax
import jax.numpy as jnp
from jax import lax
from jax.experimental import pallas as pl
from jax.experimental.pallas import tpu as pltpu
from jax.experimental.pallas import tpu_sc as plsc

_B = 16384
_D = 32
_N = 1000000
_NC = 2
_NS = 16
_NW = _NC * _NS
_BPW = _B // _NW
_CH = 16
_NCHUNK = _BPW // _CH


def _body(idx_hbm, sp_hbm, su_hbm, out_sp, out_su,
          idx_v, t_v, s_v, staged, out_buf, sem0, sem1):
    wid = lax.axis_index("s") * _NC + lax.axis_index("c")
    base = wid * _BPW
    pltpu.sync_copy(idx_hbm.at[pl.ds(base, _BPW)], idx_v)

    for g in range(_BPW // 16):
        v = idx_v[pl.ds(g * 16, 16)]
        t_v[pl.ds(g * 16, 16)] = lax.shift_right_logical(v, 3)
        s_v[pl.ds(g * 16, 16)] = lax.bitwise_and(v, 7)

    lanes = lax.iota(jnp.int32, 16)
    sems = (sem0, sem1)

    for table, out in ((sp_hbm, out_sp), (su_hbm, out_su)):
        def issue(c, buf, table=table):
            vec = t_v[pl.ds(c * _CH, _CH)]
            for k in range(_CH):
                pltpu.async_copy(
                    table.at[pl.ds(vec[k], 1)],
                    staged.at[pl.ds(buf * _CH + k, 1)],
                    sems[buf])

        def drain(buf, table=table):
            for k in range(_CH):
                pltpu.make_async_copy(
                    table.at[pl.ds(0, 1)],
                    staged.at[pl.ds(buf * _CH + k, 1)],
                    sems[buf]).wait()

        issue(0, 0)

        def chunk_body(c, carry, issue=issue, drain=drain):
            @pl.when(c + 1 < _NCHUNK)
            def _():
                lax.switch(lax.rem(c + 1, 2),
                           [lambda: issue(c + 1, 0), lambda: issue(c + 1, 1)])

            def de(buf):
                drain(buf)
                svec = s_v[pl.ds(c * _CH, _CH)]
                for k in range(_CH):
                    kv = jnp.full((16,), buf * _CH + k, jnp.int32)
                    sv = jnp.full((16,), svec[k], jnp.int32)
                    rv = jnp.full((16,), c * _CH + k, jnp.int32)
                    lo = plsc.load_gather(staged, [kv, sv, lanes])
                    hi = plsc.load_gather(staged, [kv, sv, lanes + 16])
                    plsc.store_scatter(out_buf, [rv, lanes], lo)
                    plsc.store_scatter(out_buf, [rv, lanes + 16], hi)

            lax.switch(lax.rem(c, 2), [lambda: de(0), lambda: de(1)])
            return carry

        lax.fori_loop(0, _NCHUNK, chunk_body, 0)
        pltpu.sync_copy(out_buf, out.at[pl.ds(base, _BPW)])


@jax.jit
def kernel(node_indices, B_sp, B_su):
    gather = pl.kernel(
        _body,
        out_type=(
            jax.ShapeDtypeStruct((_B, _D), jnp.float32),
            jax.ShapeDtypeStruct((_B, _D), jnp.float32),
        ),
        mesh=plsc.VectorSubcoreMesh(core_axis_name="c", subcore_axis_name="s"),
        scratch_types=[
            pltpu.VMEM((_BPW,), jnp.int32),
            pltpu.VMEM((_BPW,), jnp.int32),
            pltpu.VMEM((_BPW,), jnp.int32),
            pltpu.VMEM((2 * _CH, 8, _D), jnp.float32),
            pltpu.VMEM((_BPW, _D), jnp.float32),
            pltpu.SemaphoreType.DMA,
            pltpu.SemaphoreType.DMA,
        ],
        compiler_params=pltpu.CompilerParams(
            use_tc_tiling_on_sc=True, needs_layout_passes=False),
    )
    sp3 = B_sp.reshape(_N // 8, 8, _D)
    su3 = B_su.reshape(_N // 8, 8, _D)
    return gather(node_indices.astype(jnp.int32), sp3, su3)
